# baseline (speedup 1.0000x reference)
import functools

import jax
import jax.numpy as jnp
from jax import lax
from jax.experimental import pallas as pl
from jax.experimental.pallas import tpu as pltpu
from jax.experimental.pallas import tpu_sc as plsc

_PAD_VAL = -999.0
_L = 16


def _coef_body(e_ref, w_ref, b_ref, o_ref, *, scale):
    e = e_ref[...]
    a = jnp.dot(e, w_ref[...], preferred_element_type=jnp.float32)
    a = (a + b_ref[0:1, :]) * scale
    mask = e[:, 0:1] == _PAD_VAL
    o_ref[...] = jnp.where(mask, 0.0, a)


def _edge_coefs(edges2d, Wcat, bb, scale, block_e):
    E, DE = edges2d.shape
    KD = Wcat.shape[1]
    return pl.pallas_call(
        functools.partial(_coef_body, scale=scale),
        grid=(E // block_e,),
        in_specs=[
            pl.BlockSpec((block_e, DE), lambda i: (i, 0)),
            pl.BlockSpec((DE, KD), lambda i: (0, 0)),
            pl.BlockSpec((8, KD), lambda i: (0, 0)),
        ],
        out_specs=pl.BlockSpec((block_e, KD), lambda i: (i, 0)),
        out_shape=jax.ShapeDtypeStruct((E, KD), jnp.float32),
    )(edges2d, Wcat, bb)


def _mp_body(a_hbm, x_hbm, srci_hbm, dsti_hbm, out_hbm,
             a_v, xg_v, ax_v, idxg_v, idxs_v, stage_v, m_sh,
             sem_m, sem_x,
             *, E, NPAD, D, K, C, EPT, NCHUNK, RPT, ZR):
    cid = lax.axis_index("c")
    sid = lax.axis_index("s")
    nj = D // _L
    lanes = jnp.arange(_L, dtype=jnp.int32)

    gd = lax.GatherDimensionNumbers(
        offset_dims=(), collapsed_slice_dims=(0,), start_index_map=(0,))

    def tap(s, e_row, off):
        if 0 <= off and off + _L <= D:
            return xg_v[s][e_row, pl.ds(off, _L)]
        base = max(0, min(off, D - _L))
        v = xg_v[s][e_row, pl.ds(base, _L)]
        rel = jnp.clip(lanes + (off - base), 0, _L - 1)
        w = lax.gather(v, rel[:, None], gd, slice_sizes=(1,),
                       mode=lax.GatherScatterMode.PROMISE_IN_BOUNDS)
        pos = lanes + off
        return jnp.where((pos >= 0) & (pos < D), w, 0.0)

    def zrow(i, carry):
        for j in range(nj):
            stage_v[i, pl.ds(_L * j, _L)] = jnp.zeros((_L,), jnp.float32)
        return carry
    lax.fori_loop(0, ZR, zrow, 0)
    base_row = sid * RPT
    for r in range(RPT // ZR):
        pltpu.sync_copy(stage_v, m_sh.at[pl.ds(base_row + r * ZR, ZR)])
    plsc.subcore_barrier()

    ebase = (cid * 16 + sid) * EPT

    def fire_meta(s, g):
        b = jnp.minimum(ebase + g * C, E - C)
        pltpu.async_copy(srci_hbm.at[pl.ds(2 * b, 2 * C)], idxg_v[s],
                         sem_m[s])
        pltpu.async_copy(dsti_hbm.at[pl.ds(2 * b, 2 * C)], idxs_v[s],
                         sem_m[s])
        pltpu.async_copy(a_hbm.at[pl.ds(b * (K * D), C * K * D)], a_v[s],
                         sem_m[s])

    def wait_meta(s):
        pltpu.make_async_copy(srci_hbm.at[pl.ds(0, 2 * C)], idxg_v[s],
                              sem_m[s]).wait()
        pltpu.make_async_copy(dsti_hbm.at[pl.ds(0, 2 * C)], idxs_v[s],
                              sem_m[s]).wait()
        pltpu.make_async_copy(a_hbm.at[pl.ds(0, C * K * D)], a_v[s],
                              sem_m[s]).wait()

    def fire_gather(s):
        pltpu.async_copy(x_hbm.at[idxg_v[s]], xg_v[s], sem_x[s])

    def wait_gather(s):
        pltpu.make_async_copy(x_hbm.at[idxg_v[s]], xg_v[s],
                              sem_x[s]).wait()

    def compute(s):
        def edge(e, ecarry):
            for j in range(nj):
                accf = jnp.zeros((_L,), jnp.float32)
                accr = jnp.zeros((_L,), jnp.float32)
                for k in range(K):
                    av = a_v[s][pl.ds(e * (K * D) + k * D + _L * j, _L)]
                    off = _L * j + k - (K // 2)
                    accf = accf + av * tap(s, 2 * e, off)
                    accr = accr + av * tap(s, 2 * e + 1, off)
                ax_v[s][2 * e, pl.ds(_L * j, _L)] = accf
                ax_v[s][2 * e + 1, pl.ds(_L * j, _L)] = accr
            return ecarry
        lax.fori_loop(0, C, edge, 0)
        pltpu.sync_copy(ax_v[s], m_sh.at[idxs_v[s]], add=True)

    fire_meta(0, 0)
    fire_meta(1, 1)
    wait_meta(0)
    fire_gather(0)

    def pair(t, carry):
        i0 = 2 * t
        for s in (0, 1):
            i = i0 + s
            nxt = 1 - s
            wait_meta(nxt)
            fire_gather(nxt)
            wait_gather(s)
            compute(s)
            fire_meta(s, i + 2)
        return carry
    lax.fori_loop(0, NCHUNK // 2, pair, 0)
    wait_gather(0)
    wait_meta(1)

    plsc.subcore_barrier()
    for r in range(RPT // ZR):
        pltpu.sync_copy(m_sh.at[pl.ds(base_row + r * ZR, ZR)], stage_v)
        pltpu.sync_copy(
            stage_v, out_hbm.at[pl.ds(cid * NPAD + base_row + r * ZR, ZR)])


def _sc_message(A, x2d, srci, dsti, NPAD, D, K):
    E = A.shape[0]
    assert A.shape[1] == K * D
    C = 20
    EPT = E // 32
    NCHUNK = EPT // C
    RPT = NPAD // 16
    ZR = 40
    assert EPT * 32 == E and NCHUNK * C == EPT and RPT * 16 == NPAD
    assert (RPT % ZR) == 0 and 2 * C <= 128 and NCHUNK % 2 == 0

    mesh = plsc.VectorSubcoreMesh(core_axis_name="c", subcore_axis_name="s")
    body = functools.partial(
        _mp_body, E=E, NPAD=NPAD, D=D, K=K, C=C, EPT=EPT,
        NCHUNK=NCHUNK, RPT=RPT, ZR=ZR)
    kfn = pl.kernel(
        body,
        out_type=jax.ShapeDtypeStruct((2 * NPAD, D), jnp.float32),
        mesh=mesh,
        scratch_types=[
            [pltpu.VMEM((C * K * D,), jnp.float32)] * 2,
            [pltpu.VMEM((2 * C, D), jnp.float32)] * 2,
            [pltpu.VMEM((2 * C, D), jnp.float32)] * 2,
            [pltpu.VMEM((2 * C,), jnp.int32)] * 2,
            [pltpu.VMEM((2 * C,), jnp.int32)] * 2,
            pltpu.VMEM((ZR, D), jnp.float32),
            pltpu.VMEM_SHARED((NPAD, D), jnp.float32),
            [pltpu.SemaphoreType.DMA] * 2,
            [pltpu.SemaphoreType.DMA] * 2,
        ],
    )
    return kfn(A.reshape(-1), x2d, srci, dsti)


def _add_body(a_ref, b_ref, o_ref):
    o_ref[...] = a_ref[...] + b_ref[...]


def _combine(partials, N, NPAD, D, block_n):
    nb = N // block_n
    off = NPAD // block_n
    return pl.pallas_call(
        _add_body,
        grid=(nb,),
        in_specs=[
            pl.BlockSpec((block_n, D), lambda i: (i, 0)),
            pl.BlockSpec((block_n, D), lambda i, _o=off: (i + _o, 0)),
        ],
        out_specs=pl.BlockSpec((block_n, D), lambda i: (i, 0)),
        out_shape=jax.ShapeDtypeStruct((N, D), jnp.float32),
    )(partials, partials)


def kernel(x, edges, pairs_idx, W_enn, b_enn):
    B, N, D = x.shape
    _, E, DE = edges.shape
    K = W_enn.shape[1] // D
    assert B == 1 and K == 4 and D % _L == 0

    scale = 1.0 / (K ** 0.5)
    edges2d = edges.reshape(E, DE)
    Wcat = W_enn.reshape(DE, D, K).transpose(0, 2, 1).reshape(DE, K * D)
    bcat = b_enn.reshape(D, K).T.reshape(K * D)
    bb = jnp.broadcast_to(bcat.reshape(1, K * D), (8, K * D))
    p0 = pairs_idx[0, :, 0]
    p1 = pairs_idx[0, :, 1]
    srci = jnp.stack([p1, p0], axis=1).reshape(2 * E)
    dsti = jnp.stack([p0, p1], axis=1).reshape(2 * E)

    NPAD = 10240
    assert N <= NPAD

    A = _edge_coefs(edges2d, Wcat, bb, scale, block_e=2000)
    partials = _sc_message(A, x[0], srci, dsti, NPAD, D, K)
    m = _combine(partials, N, NPAD, D, block_n=80)
    return m.reshape(B, N, D)


# speedup vs baseline: 1.0933x; 1.0933x over previous
import functools

import jax
import jax.numpy as jnp
from jax import lax
from jax.experimental import pallas as pl
from jax.experimental.pallas import tpu as pltpu
from jax.experimental.pallas import tpu_sc as plsc

_PAD_VAL = -999.0
_L = 16


def _coef_body(e_ref, w_ref, b_ref, o_ref, *, scale):
    e = e_ref[...]
    a = jnp.dot(e, w_ref[...], preferred_element_type=jnp.float32)
    a = (a + b_ref[0:1, :]) * scale
    mask = e[:, 0:1] == _PAD_VAL
    o_ref[...] = jnp.where(mask, 0.0, a)


def _edge_coefs(edges2d, Wcat, bb, scale, block_e):
    E, DE = edges2d.shape
    KD = Wcat.shape[1]
    return pl.pallas_call(
        functools.partial(_coef_body, scale=scale),
        grid=(E // block_e,),
        in_specs=[
            pl.BlockSpec((block_e, DE), lambda i: (i, 0)),
            pl.BlockSpec((DE, KD), lambda i: (0, 0)),
            pl.BlockSpec((8, KD), lambda i: (0, 0)),
        ],
        out_specs=pl.BlockSpec((block_e, KD), lambda i: (i, 0)),
        out_shape=jax.ShapeDtypeStruct((E, KD), jnp.float32),
    )(edges2d, Wcat, bb)


def _mp_body(a_hbm, x_hbm, srci_hbm, dsti_hbm, out_hbm,
             a_v, xg_v, ax_v, idxg_v, idxs_v, stage_v, m_sh,
             sem_m, sem_x, sem_s, sem_d,
             *, E, NPAD, D, K, C, EPT, NCHUNK, RPT, ZR):
    cid = lax.axis_index("c")
    sid = lax.axis_index("s")
    nj = D // _L
    lanes = jnp.arange(_L, dtype=jnp.int32)

    gd = lax.GatherDimensionNumbers(
        offset_dims=(), collapsed_slice_dims=(0,), start_index_map=(0,))

    def tap(e_row, off):
        if 0 <= off and off + _L <= D:
            return xg_v[e_row, pl.ds(off, _L)]
        base = max(0, min(off, D - _L))
        v = xg_v[e_row, pl.ds(base, _L)]
        rel = jnp.clip(lanes + (off - base), 0, _L - 1)
        w = lax.gather(v, rel[:, None], gd, slice_sizes=(1,),
                       mode=lax.GatherScatterMode.PROMISE_IN_BOUNDS)
        pos = lanes + off
        return jnp.where((pos >= 0) & (pos < D), w, 0.0)

    def zrow(i, carry):
        for j in range(nj):
            stage_v[i, pl.ds(_L * j, _L)] = jnp.zeros((_L,), jnp.float32)
        return carry
    lax.fori_loop(0, ZR, zrow, 0)
    base_row = sid * RPT
    zcopies = [
        pltpu.async_copy(
            stage_v, m_sh.at[pl.ds(base_row + r * ZR, ZR)], sem_x)
        for r in range(RPT // ZR)]
    for cp in zcopies:
        cp.wait()
    plsc.subcore_barrier()

    ebase = (cid * 16 + sid) * EPT

    def chunk(g, carry):
        b = ebase + g * C
        pltpu.async_copy(srci_hbm.at[pl.ds(2 * b, 2 * C)], idxg_v, sem_m)
        pltpu.async_copy(a_hbm.at[pl.ds(b, C)], a_v, sem_m)
        pltpu.make_async_copy(srci_hbm.at[pl.ds(0, 2 * C)], idxg_v,
                              sem_m).wait()
        pltpu.make_async_copy(a_hbm.at[pl.ds(0, C)], a_v, sem_m).wait()
        pltpu.async_copy(x_hbm.at[idxg_v], xg_v, sem_x).wait()

        @pl.when(g > 0)
        def _():
            pltpu.make_async_copy(ax_v, m_sh.at[idxs_v], sem_s).wait()
        pltpu.async_copy(dsti_hbm.at[pl.ds(2 * b, 2 * C)], idxs_v, sem_d)

        def edge(e, ecarry):
            for j in range(nj):
                accf = accr = None
                for k in range(K):
                    av = a_v[e, pl.ds(k * D + _L * j, _L)]
                    off = _L * j + k - (K // 2)
                    pf = av * tap(2 * e, off)
                    pr = av * tap(2 * e + 1, off)
                    accf = pf if accf is None else accf + pf
                    accr = pr if accr is None else accr + pr
                ax_v[2 * e, pl.ds(_L * j, _L)] = accf
                ax_v[2 * e + 1, pl.ds(_L * j, _L)] = accr
            return ecarry
        lax.fori_loop(0, C, edge, 0)

        pltpu.make_async_copy(dsti_hbm.at[pl.ds(0, 2 * C)], idxs_v,
                              sem_d).wait()
        pltpu.async_copy(ax_v, m_sh.at[idxs_v], sem_s, add=True)
        return carry
    lax.fori_loop(0, NCHUNK, chunk, 0)
    pltpu.make_async_copy(ax_v, m_sh.at[idxs_v], sem_s).wait()

    plsc.subcore_barrier()
    pltpu.sync_copy(m_sh.at[pl.ds(base_row, RPT)],
                    out_hbm.at[pl.ds(cid * NPAD + base_row, RPT)])


def _sc_message(A, x2d, srci, dsti, NPAD, D, K):
    E = A.shape[0]
    assert A.shape[1] == K * D
    C = 40
    EPT = E // 32
    NCHUNK = EPT // C
    RPT = NPAD // 16
    ZR = 40
    assert EPT * 32 == E and NCHUNK * C == EPT and RPT * 16 == NPAD
    assert (RPT % ZR) == 0 and 2 * C <= 128 and C % 8 == 0

    mesh = plsc.VectorSubcoreMesh(core_axis_name="c", subcore_axis_name="s")
    body = functools.partial(
        _mp_body, E=E, NPAD=NPAD, D=D, K=K, C=C, EPT=EPT,
        NCHUNK=NCHUNK, RPT=RPT, ZR=ZR)
    kfn = pl.kernel(
        body,
        out_type=jax.ShapeDtypeStruct((2 * NPAD, D), jnp.float32),
        mesh=mesh,
        compiler_params=pltpu.CompilerParams(use_tc_tiling_on_sc=True),
        scratch_types=[
            pltpu.VMEM((C, K * D), jnp.float32),
            pltpu.VMEM((2 * C, D), jnp.float32),
            pltpu.VMEM((2 * C, D), jnp.float32),
            pltpu.VMEM((2 * C,), jnp.int32),
            pltpu.VMEM((2 * C,), jnp.int32),
            pltpu.VMEM((ZR, D), jnp.float32),
            pltpu.VMEM_SHARED((NPAD, D), jnp.float32),
            pltpu.SemaphoreType.DMA,
            pltpu.SemaphoreType.DMA,
            pltpu.SemaphoreType.DMA,
            pltpu.SemaphoreType.DMA,
        ],
    )
    return kfn(A, x2d, srci, dsti)


def _add_body(a_ref, b_ref, o_ref):
    o_ref[...] = a_ref[...] + b_ref[...]


def _combine(partials, N, NPAD, D, block_n):
    nb = N // block_n
    off = NPAD // block_n
    return pl.pallas_call(
        _add_body,
        grid=(nb,),
        in_specs=[
            pl.BlockSpec((block_n, D), lambda i: (i, 0)),
            pl.BlockSpec((block_n, D), lambda i, _o=off: (i + _o, 0)),
        ],
        out_specs=pl.BlockSpec((block_n, D), lambda i: (i, 0)),
        out_shape=jax.ShapeDtypeStruct((N, D), jnp.float32),
    )(partials, partials)


def kernel(x, edges, pairs_idx, W_enn, b_enn):
    B, N, D = x.shape
    _, E, DE = edges.shape
    K = W_enn.shape[1] // D
    assert B == 1 and K == 4 and D % _L == 0

    scale = 1.0 / (K ** 0.5)
    edges2d = edges.reshape(E, DE)
    Wcat = W_enn.reshape(DE, D, K).transpose(0, 2, 1).reshape(DE, K * D)
    bcat = b_enn.reshape(D, K).T.reshape(K * D)
    bb = jnp.broadcast_to(bcat.reshape(1, K * D), (8, K * D))
    p0 = pairs_idx[0, :, 0]
    p1 = pairs_idx[0, :, 1]
    srci = jnp.stack([p1, p0], axis=1).reshape(2 * E)
    dsti = jnp.stack([p0, p1], axis=1).reshape(2 * E)

    NPAD = 10240
    assert N <= NPAD

    A = _edge_coefs(edges2d, Wcat, bb, scale, block_e=2000)
    partials = _sc_message(A, x[0], srci, dsti, NPAD, D, K)
    m = _combine(partials, N, NPAD, D, block_n=80)
    return m.reshape(B, N, D)


# speedup vs baseline: 1.7084x; 1.5627x over previous
import functools

import jax
import jax.numpy as jnp
from jax import lax
from jax.experimental import pallas as pl
from jax.experimental.pallas import tpu as pltpu
from jax.experimental.pallas import tpu_sc as plsc

_PAD_VAL = -999.0
_L = 16


def _coef_body(e_ref, w_ref, b_ref, o_ref, *, scale, half):
    e = e_ref[...]
    a = jnp.dot(e, w_ref[...], preferred_element_type=jnp.float32)
    a = (a + b_ref[0:1, :]) * scale
    mask = e[:, 0:1] == _PAD_VAL
    a = jnp.where(mask, 0.0, a)
    u = lax.bitcast_convert_type(a, jnp.uint32) + jnp.uint32(0x8000)
    lo = u[:, :half] >> 16
    hi = u[:, half:] & jnp.uint32(0xFFFF0000)
    o_ref[...] = lax.bitcast_convert_type(lo | hi, jnp.int32)


def _edge_coefs(edges2d, Wcat, bb, scale, block_e):
    E, DE = edges2d.shape
    KD2 = Wcat.shape[1]
    half = KD2 // 2
    return pl.pallas_call(
        functools.partial(_coef_body, scale=scale, half=half),
        grid=(E // block_e,),
        in_specs=[
            pl.BlockSpec((block_e, DE), lambda i: (i, 0)),
            pl.BlockSpec((DE, KD2), lambda i: (0, 0)),
            pl.BlockSpec((8, KD2), lambda i: (0, 0)),
        ],
        out_specs=pl.BlockSpec((block_e, half), lambda i: (i, 0)),
        out_shape=jax.ShapeDtypeStruct((E, half), jnp.int32),
    )(edges2d, Wcat, bb)


def _mp_body(a_hbm, x_hbm, pairs_hbm, out_hbm,
             a_v, xg_v, ax_v, idxg_v, m_sh,
             sem_m, sem_x, sem_s,
             *, E, NPAD, D, K, C, EPT, NCHUNK, RPT):
    cid = lax.axis_index("c")
    sid = lax.axis_index("s")
    nj = D // _L
    lanes = jnp.arange(_L, dtype=jnp.int32)

    gd = lax.GatherDimensionNumbers(
        offset_dims=(), collapsed_slice_dims=(0,), start_index_map=(0,))

    def tap(e_row, off):
        if 0 <= off and off + _L <= D:
            return xg_v[e_row, pl.ds(off, _L)]
        base = max(0, min(off, D - _L))
        v = xg_v[e_row, pl.ds(base, _L)]
        rel = jnp.clip(lanes + (off - base), 0, _L - 1)
        w = lax.gather(v, rel[:, None], gd, slice_sizes=(1,),
                       mode=lax.GatherScatterMode.PROMISE_IN_BOUNDS)
        pos = lanes + off
        return jnp.where((pos >= 0) & (pos < D), w, 0.0)

    def zrow(i, carry):
        for j in range(nj):
            ax_v[i, pl.ds(_L * j, _L)] = jnp.zeros((_L,), jnp.float32)
        return carry
    lax.fori_loop(0, 2 * C, zrow, 0)
    base_row = sid * RPT
    zcopies = [
        pltpu.async_copy(
            ax_v, m_sh.at[pl.ds(base_row + r * 2 * C, 2 * C)], sem_x)
        for r in range(RPT // (2 * C))]
    for cp in zcopies:
        cp.wait()
    plsc.subcore_barrier()

    ebase = (cid * 16 + sid) * EPT

    def fire_meta(sl, g):
        b = ebase + g * C
        pltpu.async_copy(pairs_hbm.at[pl.ds(2 * b, 2 * C)], idxg_v[sl],
                         sem_m[sl])
        pltpu.async_copy(a_hbm.at[pl.ds(b, C)], a_v[sl], sem_m[sl])

    def wait_meta(sl):
        pltpu.make_async_copy(pairs_hbm.at[pl.ds(0, 2 * C)], idxg_v[sl],
                              sem_m[sl]).wait()
        pltpu.make_async_copy(a_hbm.at[pl.ds(0, C)], a_v[sl],
                              sem_m[sl]).wait()

    def wait_scatter(sl):
        pltpu.make_async_copy(ax_v, m_sh.at[idxg_v[sl]], sem_s).wait()

    def chunk_work(sl, g, prefetch, guard=True):
        wait_meta(sl)
        gcp = pltpu.async_copy(x_hbm.at[idxg_v[sl]], xg_v, sem_x)
        if guard:
            @pl.when(g > 0)
            def _():
                wait_scatter(1 - sl)
        else:
            wait_scatter(1 - sl)
        if prefetch:
            fire_meta(1 - sl, g + 1)
        gcp.wait()

        def edge(e, ecarry):
            for jj in range(nj // 2):
                acc = [None] * 4
                for k in range(K):
                    w32 = a_v[sl][e, pl.ds((k * (nj // 2) + jj) * _L, _L)]
                    alo = lax.bitcast_convert_type(
                        w32 << 16, jnp.float32)
                    ahi = lax.bitcast_convert_type(
                        w32 & jnp.int32(-65536), jnp.float32)
                    for h, av in ((0, alo), (1, ahi)):
                        off = _L * (2 * jj + h) + k - (K // 2)
                        pf = av * tap(2 * e + 1, off)
                        pr = av * tap(2 * e, off)
                        i0 = 2 * h
                        acc[i0] = pf if acc[i0] is None else acc[i0] + pf
                        acc[i0 + 1] = (pr if acc[i0 + 1] is None
                                       else acc[i0 + 1] + pr)
                for h in (0, 1):
                    ax_v[2 * e, pl.ds(_L * (2 * jj + h), _L)] = acc[2 * h]
                    ax_v[2 * e + 1,
                         pl.ds(_L * (2 * jj + h), _L)] = acc[2 * h + 1]
            return ecarry
        lax.fori_loop(0, C, edge, 0)
        pltpu.async_copy(ax_v, m_sh.at[idxg_v[sl]], sem_s, add=True)

    fire_meta(0, 0)

    def pair(t, carry):
        g0 = 2 * t
        for sl in (0, 1):
            chunk_work(sl, g0 + sl, prefetch=True)
        return carry
    lax.fori_loop(0, (NCHUNK - 1) // 2, pair, 0)
    chunk_work(0, NCHUNK - 1, prefetch=False, guard=False)
    wait_scatter(0)

    plsc.subcore_barrier()
    pltpu.sync_copy(m_sh.at[pl.ds(base_row, RPT)],
                    out_hbm.at[pl.ds(cid * NPAD + base_row, RPT)])


def _sc_message(A, x2d, pairs1d, NPAD, D, K):
    E = A.shape[0]
    assert A.shape[1] == K * D // 2
    C = 40
    EPT = E // 32
    NCHUNK = EPT // C
    RPT = NPAD // 16
    assert EPT * 32 == E and NCHUNK * C == EPT and RPT * 16 == NPAD
    assert RPT % (2 * C) == 0 and 2 * C <= 128 and C % 8 == 0
    assert NCHUNK % 2 == 1

    mesh = plsc.VectorSubcoreMesh(core_axis_name="c", subcore_axis_name="s")
    body = functools.partial(
        _mp_body, E=E, NPAD=NPAD, D=D, K=K, C=C, EPT=EPT,
        NCHUNK=NCHUNK, RPT=RPT)
    kfn = pl.kernel(
        body,
        out_type=jax.ShapeDtypeStruct((2 * NPAD, D), jnp.float32),
        mesh=mesh,
        compiler_params=pltpu.CompilerParams(use_tc_tiling_on_sc=True),
        scratch_types=[
            [pltpu.VMEM((C, K * D // 2), jnp.int32)] * 2,
            pltpu.VMEM((2 * C, D), jnp.float32),
            pltpu.VMEM((2 * C, D), jnp.float32),
            [pltpu.VMEM((2 * C,), jnp.int32)] * 2,
            pltpu.VMEM_SHARED((NPAD, D), jnp.float32),
            [pltpu.SemaphoreType.DMA] * 2,
            pltpu.SemaphoreType.DMA,
            pltpu.SemaphoreType.DMA,
        ],
    )
    return kfn(A, x2d, pairs1d)


def _add_body(a_ref, b_ref, o_ref):
    o_ref[...] = a_ref[...] + b_ref[...]


def _combine(partials, N, NPAD, D, block_n):
    nb = N // block_n
    off = NPAD // block_n
    return pl.pallas_call(
        _add_body,
        grid=(nb,),
        in_specs=[
            pl.BlockSpec((block_n, D), lambda i: (i, 0)),
            pl.BlockSpec((block_n, D), lambda i, _o=off: (i + _o, 0)),
        ],
        out_specs=pl.BlockSpec((block_n, D), lambda i: (i, 0)),
        out_shape=jax.ShapeDtypeStruct((N, D), jnp.float32),
    )(partials, partials)


def kernel(x, edges, pairs_idx, W_enn, b_enn):
    B, N, D = x.shape
    _, E, DE = edges.shape
    K = W_enn.shape[1] // D
    assert B == 1 and K == 4 and D % _L == 0

    scale = 1.0 / (K ** 0.5)
    edges2d = edges.reshape(E, DE)
    import numpy as np
    kk, jj, w = np.meshgrid(np.arange(K), np.arange(D // 32),
                            np.arange(_L), indexing="ij")
    d_lo = (32 * jj + w).reshape(-1)
    d_hi = d_lo + _L
    k_f = kk.reshape(-1)
    cols = np.concatenate([4 * d_lo + k_f, 4 * d_hi + k_f])
    Wcat = W_enn[:, cols]
    bcat = b_enn[cols]
    bb = jnp.broadcast_to(bcat.reshape(1, -1), (8, cols.size))
    pairs1d = pairs_idx[0].reshape(2 * E)

    NPAD = 10240
    assert N <= NPAD

    A = _edge_coefs(edges2d, Wcat, bb, scale, block_e=2000)
    partials = _sc_message(A, x[0], pairs1d, NPAD, D, K)
    m = _combine(partials, N, NPAD, D, block_n=80)
    return m.reshape(B, N, D)


# speedup vs baseline: 1.8190x; 1.0647x over previous
import functools

import jax
import jax.numpy as jnp
from jax import lax
from jax.experimental import pallas as pl
from jax.experimental.pallas import tpu as pltpu
from jax.experimental.pallas import tpu_sc as plsc

_PAD_VAL = -999.0
_L = 16


def _coef_body(e_ref, w_ref, b_ref, o_ref, *, scale, half):
    e = e_ref[...]
    a = jnp.dot(e, w_ref[...], preferred_element_type=jnp.float32)
    a = (a + b_ref[0:1, :]) * scale
    mask = e[:, 0:1] == _PAD_VAL
    a = jnp.where(mask, 0.0, a)
    u = lax.bitcast_convert_type(a, jnp.uint32) + jnp.uint32(0x8000)
    lo = u[:, :half] >> 16
    hi = u[:, half:] & jnp.uint32(0xFFFF0000)
    o_ref[...] = lax.bitcast_convert_type(lo | hi, jnp.int32)


def _edge_coefs(edges2d, Wcat, bb, scale, block_e):
    E, DE = edges2d.shape
    KD2 = Wcat.shape[1]
    half = KD2 // 2
    return pl.pallas_call(
        functools.partial(_coef_body, scale=scale, half=half),
        grid=(E // block_e,),
        in_specs=[
            pl.BlockSpec((block_e, DE), lambda i: (i, 0)),
            pl.BlockSpec((DE, KD2), lambda i: (0, 0)),
            pl.BlockSpec((8, KD2), lambda i: (0, 0)),
        ],
        out_specs=pl.BlockSpec((block_e, half), lambda i: (i, 0)),
        out_shape=jax.ShapeDtypeStruct((E, half), jnp.int32),
    )(edges2d, Wcat, bb)


def _mp_body(a_hbm, x_hbm, pairs_hbm, out_hbm,
             a_v, xg_v, ax_v, idxg_v, m_sh,
             sem_m, sem_x, sem_s,
             *, E, NPAD, D, K, C, EPT, NCHUNK, RPT):
    cid = lax.axis_index("c")
    sid = lax.axis_index("s")
    nj = D // _L
    lanes = jnp.arange(_L, dtype=jnp.int32)

    gd = lax.GatherDimensionNumbers(
        offset_dims=(), collapsed_slice_dims=(0,), start_index_map=(0,))

    def tap(e_row, off):
        if 0 <= off and off + _L <= D:
            return xg_v[e_row, pl.ds(off, _L)]
        base = max(0, min(off, D - _L))
        v = xg_v[e_row, pl.ds(base, _L)]
        rel = jnp.clip(lanes + (off - base), 0, _L - 1)
        w = lax.gather(v, rel[:, None], gd, slice_sizes=(1,),
                       mode=lax.GatherScatterMode.PROMISE_IN_BOUNDS)
        pos = lanes + off
        return jnp.where((pos >= 0) & (pos < D), w, 0.0)

    def zrow(i, carry):
        for j in range(nj):
            ax_v[i, pl.ds(_L * j, _L)] = jnp.zeros((_L,), jnp.float32)
        return carry
    lax.fori_loop(0, 2 * C, zrow, 0)
    base_row = sid * RPT
    zcopies = [
        pltpu.async_copy(
            ax_v, m_sh.at[pl.ds(base_row + r * 2 * C, 2 * C)], sem_x)
        for r in range(RPT // (2 * C))]
    for cp in zcopies:
        cp.wait()
    plsc.subcore_barrier()

    ebase = (cid * 16 + sid) * EPT

    def fire_meta(sl, g):
        b = ebase + g * C
        pltpu.async_copy(pairs_hbm.at[pl.ds(2 * b, 2 * C)], idxg_v[sl],
                         sem_m[sl])
        pltpu.async_copy(a_hbm.at[pl.ds(b, C)], a_v[sl], sem_m[sl])

    def wait_meta(sl):
        pltpu.make_async_copy(pairs_hbm.at[pl.ds(0, 2 * C)], idxg_v[sl],
                              sem_m[sl]).wait()
        pltpu.make_async_copy(a_hbm.at[pl.ds(0, C)], a_v[sl],
                              sem_m[sl]).wait()

    def wait_scatter(sl):
        pltpu.make_async_copy(ax_v, m_sh.at[idxg_v[sl]], sem_s).wait()

    def chunk_work(sl, g, prefetch, guard=True):
        wait_meta(sl)
        gcp = pltpu.async_copy(x_hbm.at[idxg_v[sl]], xg_v, sem_x)
        if guard:
            @pl.when(g > 0)
            def _():
                wait_scatter(1 - sl)
        else:
            wait_scatter(1 - sl)
        if prefetch:
            fire_meta(1 - sl, g + 1)
        gcp.wait()

        def edge(e, ecarry):
            for jj in range(nj // 2):
                acc = [None] * 4
                for k in range(K):
                    w32 = a_v[sl][e, pl.ds((k * (nj // 2) + jj) * _L, _L)]
                    alo = lax.bitcast_convert_type(
                        w32 << 16, jnp.float32)
                    ahi = lax.bitcast_convert_type(
                        w32 & jnp.int32(-65536), jnp.float32)
                    for h, av in ((0, alo), (1, ahi)):
                        off = _L * (2 * jj + h) + k - (K // 2)
                        pf = av * tap(2 * e + 1, off)
                        pr = av * tap(2 * e, off)
                        i0 = 2 * h
                        acc[i0] = pf if acc[i0] is None else acc[i0] + pf
                        acc[i0 + 1] = (pr if acc[i0 + 1] is None
                                       else acc[i0 + 1] + pr)
                for h in (0, 1):
                    ax_v[2 * e, pl.ds(_L * (2 * jj + h), _L)] = acc[2 * h]
                    ax_v[2 * e + 1,
                         pl.ds(_L * (2 * jj + h), _L)] = acc[2 * h + 1]
            return ecarry
        lax.fori_loop(0, C, edge, 0)
        pltpu.async_copy(ax_v, m_sh.at[idxg_v[sl]], sem_s, add=True)

    fire_meta(0, 0)

    def pair(t, carry):
        g0 = 2 * t
        for sl in (0, 1):
            chunk_work(sl, g0 + sl, prefetch=True)
        return carry
    lax.fori_loop(0, (NCHUNK - 1) // 2, pair, 0)
    chunk_work(0, NCHUNK - 1, prefetch=False, guard=False)
    wait_scatter(0)

    plsc.subcore_barrier()
    pltpu.sync_copy(m_sh.at[pl.ds(base_row, RPT)],
                    out_hbm.at[pl.ds((2 * sid + cid) * RPT, RPT)])


def _sc_message(A, x2d, pairs1d, NPAD, D, K):
    E = A.shape[0]
    assert A.shape[1] == K * D // 2
    C = 40
    EPT = E // 32
    NCHUNK = EPT // C
    RPT = NPAD // 16
    assert EPT * 32 == E and NCHUNK * C == EPT and RPT * 16 == NPAD
    assert RPT % (2 * C) == 0 and 2 * C <= 128 and C % 8 == 0
    assert NCHUNK % 2 == 1

    mesh = plsc.VectorSubcoreMesh(core_axis_name="c", subcore_axis_name="s")
    body = functools.partial(
        _mp_body, E=E, NPAD=NPAD, D=D, K=K, C=C, EPT=EPT,
        NCHUNK=NCHUNK, RPT=RPT)
    kfn = pl.kernel(
        body,
        out_type=jax.ShapeDtypeStruct((2 * NPAD, D), jnp.float32),
        mesh=mesh,
        compiler_params=pltpu.CompilerParams(use_tc_tiling_on_sc=True),
        scratch_types=[
            [pltpu.VMEM((C, K * D // 2), jnp.int32)] * 2,
            pltpu.VMEM((2 * C, D), jnp.float32),
            pltpu.VMEM((2 * C, D), jnp.float32),
            [pltpu.VMEM((2 * C,), jnp.int32)] * 2,
            pltpu.VMEM_SHARED((NPAD, D), jnp.float32),
            [pltpu.SemaphoreType.DMA] * 2,
            pltpu.SemaphoreType.DMA,
            pltpu.SemaphoreType.DMA,
        ],
    )
    return kfn(A, x2d, pairs1d)


def _add_body(a_ref, b_ref, o_ref):
    o_ref[...] = a_ref[...] + b_ref[...]


def _combine(partials, N, NPAD, D):
    block_n = NPAD // 16
    return pl.pallas_call(
        _add_body,
        grid=(16,),
        in_specs=[
            pl.BlockSpec((block_n, D), lambda i: (2 * i, 0)),
            pl.BlockSpec((block_n, D), lambda i: (2 * i + 1, 0)),
        ],
        out_specs=pl.BlockSpec((block_n, D), lambda i: (i, 0)),
        out_shape=jax.ShapeDtypeStruct((NPAD, D), jnp.float32),
    )(partials, partials)


def kernel(x, edges, pairs_idx, W_enn, b_enn):
    B, N, D = x.shape
    _, E, DE = edges.shape
    K = W_enn.shape[1] // D
    assert B == 1 and K == 4 and D % _L == 0

    scale = 1.0 / (K ** 0.5)
    edges2d = edges.reshape(E, DE)
    import numpy as np
    kk, jj, w = np.meshgrid(np.arange(K), np.arange(D // 32),
                            np.arange(_L), indexing="ij")
    d_lo = (32 * jj + w).reshape(-1)
    d_hi = d_lo + _L
    k_f = kk.reshape(-1)
    cols = np.concatenate([4 * d_lo + k_f, 4 * d_hi + k_f])
    Wcat = W_enn[:, cols]
    bcat = b_enn[cols]
    bb = jnp.broadcast_to(bcat.reshape(1, -1), (8, cols.size))
    pairs1d = pairs_idx[0].reshape(2 * E)

    NPAD = 10240
    assert N <= NPAD

    A = _edge_coefs(edges2d, Wcat, bb, scale, block_e=2000)
    partials = _sc_message(A, x[0], pairs1d, NPAD, D, K)
    m = _combine(partials, N, NPAD, D)
    return m[:N].reshape(B, N, D)
